# baseline (speedup 1.0000x reference)
import functools

import jax
import jax.numpy as jnp
from jax import lax
from jax.experimental import pallas as pl
from jax.experimental.pallas import tpu as pltpu
from jax.experimental.pallas import tpu_sc as plsc


@functools.lru_cache(maxsize=None)
def _make_gather(V, D, Bx, S, MC, cbase):
    info = plsc.get_sparse_core_info()
    NC, NS = info.num_cores, info.num_subcores
    NW = NC * NS
    b_per_w = MC // NW
    CH = 64
    NB = 3
    n_ch = b_per_w // CH
    mesh = plsc.VectorSubcoreMesh(core_axis_name="c", subcore_axis_name="s")

    @functools.partial(
        pl.kernel,
        mesh=mesh,
        out_type=jax.ShapeDtypeStruct((MC, D), jnp.float32),
        scratch_types=[
            pltpu.VMEM((b_per_w,), jnp.int32),
            *[pltpu.VMEM((CH, D), jnp.float32) for _ in range(NB)],
            *[pltpu.SemaphoreType.DMA for _ in range(2 * NB)],
        ],
    )
    def gather(table_hbm, idx_hbm, out_hbm, idx_v, *rest):
        bufs = rest[:NB]
        gsems = rest[NB:2 * NB]
        wsems = rest[2 * NB:]
        wid = lax.axis_index("s") * NC + lax.axis_index("c")
        base = wid * b_per_w
        g0 = cbase + base
        pltpu.sync_copy(idx_hbm.at[g0 // S, pl.ds(g0 % S, b_per_w)], idx_v)

        def g_start(c):
            return pltpu.async_copy(
                table_hbm.at[idx_v.at[pl.ds(c * CH, CH)]],
                bufs[c % NB], gsems[c % NB])

        def w_start(c):
            return pltpu.async_copy(
                bufs[c % NB], out_hbm.at[pl.ds(base + c * CH, CH)],
                wsems[c % NB])

        g = [None] * n_ch
        w = [None] * n_ch
        for c in range(min(NB, n_ch)):
            g[c] = g_start(c)
        for c in range(n_ch):
            g[c].wait()
            w[c] = w_start(c)
            if c + NB < n_ch:
                w[c].wait()
                g[c + NB] = g_start(c + NB)
        for c in range(max(0, n_ch - NB), n_ch):
            w[c].wait()

    return gather


def _mm_body(e_ref, w_ref, b_ref, o_ref):
    o_ref[0] = lax.dot_general(
        e_ref[...], w_ref[...],
        dimension_numbers=(((1,), (1,)), ((), ())),
        preferred_element_type=jnp.float32,
    ) + b_ref[...]


def _mm_body_alias(e_ref, w_ref, b_ref, _prev_ref, o_ref):
    _mm_body(e_ref, w_ref, b_ref, o_ref)


@functools.lru_cache(maxsize=None)
def _make_matmul(Bx, S, MC, D, N, c, aliased, BM=2048):
    nblk = MC // BM
    SB = S // BM
    off = c
    in_specs = [
        pl.BlockSpec((BM, D), lambda i: (i, 0)),
        pl.BlockSpec((N, D), lambda i: (0, 0)),
        pl.BlockSpec((1, N), lambda i: (0, 0)),
    ]
    body = _mm_body
    kwargs = {}
    if aliased:
        in_specs.append(pl.BlockSpec(memory_space=pl.ANY))
        body = _mm_body_alias
        kwargs["input_output_aliases"] = {3: 0}
    return pl.pallas_call(
        body,
        grid=(nblk,),
        in_specs=in_specs,
        out_specs=pl.BlockSpec(
            (1, BM, N), lambda i: ((off + i) // SB, (off + i) % SB, 0)),
        out_shape=jax.ShapeDtypeStruct((Bx, S, N), jnp.float32),
        **kwargs,
    )


_BM = 2048
_CHUNKS = (4096, 4096)


def kernel(x, table, W, b):
    Bx, S = x.shape
    V, D = table.shape
    N = W.shape[0]
    M = Bx * S
    del M
    b2 = b.reshape(1, N)
    starts = [sum(_CHUNKS[:i]) for i in range(len(_CHUNKS))]
    embs = [_make_gather(V, D, Bx, S, mc, s)(table, x)
            for mc, s in zip(_CHUNKS, starts)]
    out = None
    for i, (mc, s) in enumerate(zip(_CHUNKS, starts)):
        mm = _make_matmul(Bx, S, mc, D, N, s // _BM, i > 0, _BM)
        out = mm(embs[i], W, b2) if i == 0 else mm(embs[i], W, b2, out)
    return out


# speedup vs baseline: 1.0039x; 1.0039x over previous
import functools

import jax
import jax.numpy as jnp
from jax import lax
from jax.experimental import pallas as pl
from jax.experimental.pallas import tpu as pltpu
from jax.experimental.pallas import tpu_sc as plsc


@functools.lru_cache(maxsize=None)
def _make_gather(V, D, Bx, S, MC, cbase):
    info = plsc.get_sparse_core_info()
    NC, NS = info.num_cores, info.num_subcores
    NW = NC * NS
    b_per_w = MC // NW
    CH = 64
    NB = 3
    n_ch = b_per_w // CH
    mesh = plsc.VectorSubcoreMesh(core_axis_name="c", subcore_axis_name="s")

    @functools.partial(
        pl.kernel,
        mesh=mesh,
        out_type=jax.ShapeDtypeStruct((MC, D), jnp.float32),
        scratch_types=[
            pltpu.VMEM((b_per_w,), jnp.int32),
            *[pltpu.VMEM((CH, D), jnp.float32) for _ in range(NB)],
            *[pltpu.SemaphoreType.DMA for _ in range(2 * NB)],
        ],
    )
    def gather(table_hbm, idx_hbm, out_hbm, idx_v, *rest):
        bufs = rest[:NB]
        gsems = rest[NB:2 * NB]
        wsems = rest[2 * NB:]
        wid = lax.axis_index("s") * NC + lax.axis_index("c")
        base = wid * b_per_w
        g0 = cbase + base
        pltpu.sync_copy(idx_hbm.at[g0 // S, pl.ds(g0 % S, b_per_w)], idx_v)

        def g_start(c):
            return pltpu.async_copy(
                table_hbm.at[idx_v.at[pl.ds(c * CH, CH)]],
                bufs[c % NB], gsems[c % NB])

        def w_start(c):
            return pltpu.async_copy(
                bufs[c % NB], out_hbm.at[pl.ds(base + c * CH, CH)],
                wsems[c % NB])

        g = [None] * n_ch
        w = [None] * n_ch
        for c in range(min(NB, n_ch)):
            g[c] = g_start(c)
        for c in range(n_ch):
            g[c].wait()
            w[c] = w_start(c)
            if c + NB < n_ch:
                w[c].wait()
                g[c + NB] = g_start(c + NB)
        for c in range(max(0, n_ch - NB), n_ch):
            w[c].wait()

    return gather


def _mm_body(e_ref, w_ref, b_ref, o_ref):
    o_ref[0] = lax.dot_general(
        e_ref[...], w_ref[...],
        dimension_numbers=(((1,), (1,)), ((), ())),
        preferred_element_type=jnp.float32,
    ) + b_ref[...]


def _mm_body_alias(e_ref, w_ref, b_ref, _prev_ref, o_ref):
    _mm_body(e_ref, w_ref, b_ref, o_ref)


@functools.lru_cache(maxsize=None)
def _make_matmul(Bx, S, MC, D, N, c, aliased, BM=2048):
    nblk = MC // BM
    SB = S // BM
    off = c
    in_specs = [
        pl.BlockSpec((BM, D), lambda i: (i, 0)),
        pl.BlockSpec((N, D), lambda i: (0, 0)),
        pl.BlockSpec((1, N), lambda i: (0, 0)),
    ]
    body = _mm_body
    kwargs = {}
    if aliased:
        in_specs.append(pl.BlockSpec(memory_space=pl.ANY))
        body = _mm_body_alias
        kwargs["input_output_aliases"] = {3: 0}
    return pl.pallas_call(
        body,
        grid=(nblk,),
        in_specs=in_specs,
        out_specs=pl.BlockSpec(
            (1, BM, N), lambda i: ((off + i) // SB, (off + i) % SB, 0)),
        out_shape=jax.ShapeDtypeStruct((Bx, S, N), jnp.float32),
        **kwargs,
    )


_BM = 2048
_CHUNKS = (4096, 4096)


def kernel(x, table, W, b):
    Bx, S = x.shape
    V, D = table.shape
    N = W.shape[0]
    M = Bx * S
    del M
    x = x.astype(jnp.int32)
    b2 = b.reshape(1, N)
    starts = [sum(_CHUNKS[:i]) for i in range(len(_CHUNKS))]
    embs = [_make_gather(V, D, Bx, S, mc, s)(table, x)
            for mc, s in zip(_CHUNKS, starts)]
    out = None
    for i, (mc, s) in enumerate(zip(_CHUNKS, starts)):
        mm = _make_matmul(Bx, S, mc, D, N, s // _BM, i > 0, _BM)
        out = mm(embs[i], W, b2) if i == 0 else mm(embs[i], W, b2, out)
    return out
